# baseline (speedup 1.0000x reference)
import functools

import jax
import jax.numpy as jnp
from jax import lax
from jax.experimental import pallas as pl
from jax.experimental.pallas import tpu as pltpu
from jax.experimental.pallas import tpu_sc as plsc

SEQ_LEN = 4096
EMB = 1024
NUM_CORES = 2
NUM_SUBCORES = 16
NUM_WORKERS = NUM_CORES * NUM_SUBCORES
ROWS_PER_WORKER = SEQ_LEN // NUM_WORKERS
CHUNK = 8
NUM_CHUNKS = ROWS_PER_WORKER // CHUNK
NUM_BUFS = 8

@functools.lru_cache(maxsize=1)
def _build_copy_rows():
    mesh = plsc.VectorSubcoreMesh(
        core_axis_name="c", subcore_axis_name="s",
        num_cores=NUM_CORES, num_subcores=NUM_SUBCORES)

    @functools.partial(
        pl.kernel,
        out_type=jax.ShapeDtypeStruct((SEQ_LEN, EMB), jnp.float32),
        mesh=mesh,
        scratch_types=(
            [pltpu.VMEM((CHUNK, EMB), jnp.float32)] * NUM_BUFS
            + [pltpu.SemaphoreType.DMA] * (2 * NUM_BUFS)
        ),
    )
    def copy_rows(table_hbm, out_hbm, *scratch):
        bufs = scratch[:NUM_BUFS]
        isems = scratch[NUM_BUFS:2 * NUM_BUFS]
        osems = scratch[2 * NUM_BUFS:]
        wid = lax.axis_index("s") * NUM_CORES + lax.axis_index("c")
        base = wid * ROWS_PER_WORKER

        def in_copy(i):
            b = i % NUM_BUFS
            return pltpu.make_async_copy(
                table_hbm.at[pl.ds(base + i * CHUNK, CHUNK)], bufs[b], isems[b])

        def out_copy(i):
            b = i % NUM_BUFS
            return pltpu.make_async_copy(
                bufs[b], out_hbm.at[pl.ds(base + i * CHUNK, CHUNK)], osems[b])

        for i in range(min(NUM_BUFS, NUM_CHUNKS)):
            in_copy(i).start()
        for i in range(NUM_CHUNKS):
            in_copy(i).wait()
            out_copy(i).start()
            nxt = i + NUM_BUFS
            if nxt < NUM_CHUNKS:
                out_copy(nxt - NUM_BUFS).wait()
                in_copy(nxt).start()
        for i in range(max(0, NUM_CHUNKS - NUM_BUFS), NUM_CHUNKS):
            out_copy(i).wait()

    return copy_rows


def kernel(seq_len, pos_embedding):
    del seq_len
    return _build_copy_rows()(pos_embedding)


# speedup vs baseline: 1.0122x; 1.0122x over previous
import functools

import jax
import jax.numpy as jnp
from jax import lax
from jax.experimental import pallas as pl
from jax.experimental.pallas import tpu as pltpu
from jax.experimental.pallas import tpu_sc as plsc

SEQ_LEN = 4096
EMB = 1024
NUM_CORES = 2
NUM_SUBCORES = 16
NUM_WORKERS = NUM_CORES * NUM_SUBCORES
ROWS_PER_WORKER = SEQ_LEN // NUM_WORKERS

CHUNK_SIZES = (8, 16, 16, 16, 16, 16, 16, 16, 8)
assert sum(CHUNK_SIZES) == ROWS_PER_WORKER
CHUNK_OFFS = tuple(sum(CHUNK_SIZES[:i]) for i in range(len(CHUNK_SIZES)))
NUM_CHUNKS = len(CHUNK_SIZES)
BUF_ROWS = max(CHUNK_SIZES)
NUM_BUFS = 6


@functools.lru_cache(maxsize=1)
def _build_copy_rows():
    mesh = plsc.VectorSubcoreMesh(
        core_axis_name="c", subcore_axis_name="s",
        num_cores=NUM_CORES, num_subcores=NUM_SUBCORES)

    @functools.partial(
        pl.kernel,
        out_type=jax.ShapeDtypeStruct((SEQ_LEN, EMB), jnp.float32),
        mesh=mesh,
        scratch_types=(
            [pltpu.VMEM((BUF_ROWS, EMB), jnp.float32)] * NUM_BUFS
            + [pltpu.SemaphoreType.DMA] * (2 * NUM_BUFS)
        ),
    )
    def copy_rows(table_hbm, out_hbm, *scratch):
        bufs = scratch[:NUM_BUFS]
        isems = scratch[NUM_BUFS:2 * NUM_BUFS]
        osems = scratch[2 * NUM_BUFS:]
        wid = lax.axis_index("s") * NUM_CORES + lax.axis_index("c")
        base = wid * ROWS_PER_WORKER

        def in_copy(i):
            b = i % NUM_BUFS
            return pltpu.make_async_copy(
                table_hbm.at[pl.ds(base + CHUNK_OFFS[i], CHUNK_SIZES[i])],
                bufs[b].at[pl.ds(0, CHUNK_SIZES[i])], isems[b])

        def out_copy(i):
            b = i % NUM_BUFS
            return pltpu.make_async_copy(
                bufs[b].at[pl.ds(0, CHUNK_SIZES[i])],
                out_hbm.at[pl.ds(base + CHUNK_OFFS[i], CHUNK_SIZES[i])],
                osems[b])

        for i in range(min(NUM_BUFS, NUM_CHUNKS)):
            in_copy(i).start()
        for i in range(NUM_CHUNKS):
            in_copy(i).wait()
            out_copy(i).start()
            nxt = i + NUM_BUFS
            if nxt < NUM_CHUNKS:
                out_copy(nxt - NUM_BUFS).wait()
                in_copy(nxt).start()
        for i in range(max(0, NUM_CHUNKS - NUM_BUFS), NUM_CHUNKS):
            out_copy(i).wait()

    return copy_rows


def kernel(seq_len, pos_embedding):
    del seq_len
    return _build_copy_rows()(pos_embedding)


# speedup vs baseline: 1.0342x; 1.0217x over previous
import functools

import jax
import jax.numpy as jnp
from jax import lax
from jax.experimental import pallas as pl
from jax.experimental.pallas import tpu as pltpu
from jax.experimental.pallas import tpu_sc as plsc

SEQ_LEN = 4096
EMB = 1024
NUM_CORES = 2
NUM_SUBCORES = 16
NUM_WORKERS = NUM_CORES * NUM_SUBCORES
ROWS_PER_WORKER = SEQ_LEN // NUM_WORKERS

CHUNK_SIZES = (16,) * 8
assert sum(CHUNK_SIZES) == ROWS_PER_WORKER
CHUNK_OFFS = tuple(sum(CHUNK_SIZES[:i]) for i in range(len(CHUNK_SIZES)))
NUM_CHUNKS = len(CHUNK_SIZES)
BUF_ROWS = max(CHUNK_SIZES)
NUM_BUFS = 6


@functools.lru_cache(maxsize=1)
def _build_copy_rows():
    mesh = plsc.VectorSubcoreMesh(
        core_axis_name="c", subcore_axis_name="s",
        num_cores=NUM_CORES, num_subcores=NUM_SUBCORES)

    @functools.partial(
        pl.kernel,
        out_type=jax.ShapeDtypeStruct((SEQ_LEN, EMB), jnp.float32),
        mesh=mesh,
        scratch_types=(
            [pltpu.VMEM((BUF_ROWS, EMB), jnp.float32)] * NUM_BUFS
            + [pltpu.SemaphoreType.DMA] * (2 * NUM_BUFS)
        ),
    )
    def copy_rows(table_hbm, out_hbm, *scratch):
        bufs = scratch[:NUM_BUFS]
        isems = scratch[NUM_BUFS:2 * NUM_BUFS]
        osems = scratch[2 * NUM_BUFS:]
        wid = lax.axis_index("s") * NUM_CORES + lax.axis_index("c")
        base = wid * ROWS_PER_WORKER

        def in_copy(i):
            b = i % NUM_BUFS
            return pltpu.make_async_copy(
                table_hbm.at[pl.ds(base + CHUNK_OFFS[i], CHUNK_SIZES[i])],
                bufs[b].at[pl.ds(0, CHUNK_SIZES[i])], isems[b])

        def out_copy(i):
            b = i % NUM_BUFS
            return pltpu.make_async_copy(
                bufs[b].at[pl.ds(0, CHUNK_SIZES[i])],
                out_hbm.at[pl.ds(base + CHUNK_OFFS[i], CHUNK_SIZES[i])],
                osems[b])

        for i in range(min(NUM_BUFS, NUM_CHUNKS)):
            in_copy(i).start()
        for i in range(NUM_CHUNKS):
            in_copy(i).wait()
            out_copy(i).start()
            nxt = i + NUM_BUFS
            if nxt < NUM_CHUNKS:
                out_copy(nxt - NUM_BUFS).wait()
                in_copy(nxt).start()
        for i in range(max(0, NUM_CHUNKS - NUM_BUFS), NUM_CHUNKS):
            out_copy(i).wait()

    return copy_rows


def kernel(seq_len, pos_embedding):
    del seq_len
    return _build_copy_rows()(pos_embedding)
